# Initial kernel scaffold; baseline (speedup 1.0000x reference)
import jax
import jax.numpy as jnp
from jax.experimental import pallas as pl


def kernel(stu_id, exer_id, theta_w, a_w, b_w):
    raise NotImplementedError("write your pallas kernel here")


# speedup vs baseline: 1.1382x; 1.1382x over previous
import functools

import jax
import jax.numpy as jnp
from jax import lax
from jax.experimental import pallas as pl
from jax.experimental.pallas import tpu as pltpu
from jax.experimental.pallas import tpu_sc as plsc

BATCH = 16384
_INFO = plsc.get_sparse_core_info()
_NC, _NS, _L = _INFO.num_cores, _INFO.num_subcores, _INFO.num_lanes
_NW = _NC * _NS
_BPW = BATCH // _NW


def _irt_body(stu_hbm, exer_hbm, theta_hbm, a_hbm, b_hbm, out_hbm,
              stu_v, exer_v, th_v, a_v, b_v, out_v, sem):
    wid = lax.axis_index("s") * _NC + lax.axis_index("c")
    base = wid * _BPW
    pltpu.sync_copy(stu_hbm.at[pl.ds(base, _BPW)], stu_v)
    pltpu.sync_copy(exer_hbm.at[pl.ds(base, _BPW)], exer_v)
    c1 = pltpu.async_copy(theta_hbm.at[stu_v], th_v, sem)
    c2 = pltpu.async_copy(a_hbm.at[exer_v], a_v, sem)
    c3 = pltpu.async_copy(b_hbm.at[exer_v], b_v, sem)
    c1.wait()
    c2.wait()
    c3.wait()
    for j in range(_BPW // _L):
        sl = pl.ds(j * _L, _L)
        th = th_v[sl]
        ar = a_v[sl]
        br = b_v[sl]
        a2 = 2.0 / (1.0 + jnp.exp(-ar))
        z = 1.7 * a2 * (th - br)
        out_v[sl] = 1.0 / (1.0 + jnp.exp(-z))
    pltpu.sync_copy(out_v, out_hbm.at[pl.ds(base, _BPW)])


_irt_sc = functools.partial(
    pl.kernel,
    mesh=plsc.VectorSubcoreMesh(core_axis_name="c", subcore_axis_name="s"),
    out_type=jax.ShapeDtypeStruct((BATCH,), jnp.float32),
    scratch_types=[
        pltpu.VMEM((_BPW,), jnp.int32),
        pltpu.VMEM((_BPW,), jnp.int32),
        pltpu.VMEM((_BPW,), jnp.float32),
        pltpu.VMEM((_BPW,), jnp.float32),
        pltpu.VMEM((_BPW,), jnp.float32),
        pltpu.VMEM((_BPW,), jnp.float32),
        pltpu.SemaphoreType.DMA,
    ],
)(_irt_body)


def kernel(stu_id, exer_id, theta_w, a_w, b_w):
    return _irt_sc(
        stu_id.astype(jnp.int32),
        exer_id.astype(jnp.int32),
        theta_w.reshape(-1),
        a_w.reshape(-1),
        b_w.reshape(-1),
    )


# speedup vs baseline: 1.1491x; 1.0096x over previous
import functools

import jax
import jax.numpy as jnp
from jax import lax
from jax.experimental import pallas as pl
from jax.experimental.pallas import tpu as pltpu
from jax.experimental.pallas import tpu_sc as plsc

BATCH = 16384
_INFO = plsc.get_sparse_core_info()
_NC, _NS, _L = _INFO.num_cores, _INFO.num_subcores, _INFO.num_lanes
_NW = _NC * _NS
_BPW = BATCH // _NW


_HALF = _BPW // 2


def _irt_body(stu_hbm, exer_hbm, theta_hbm, a_hbm, b_hbm, out_hbm,
              stu_v, exer_v, th_v, a_v, b_v, out_v,
              sem_i, sem_a, sem_b, sem_o):
    wid = lax.axis_index("s") * _NC + lax.axis_index("c")
    base = wid * _BPW
    ci1 = pltpu.async_copy(stu_hbm.at[pl.ds(base, _BPW)], stu_v, sem_i)
    ci2 = pltpu.async_copy(exer_hbm.at[pl.ds(base, _BPW)], exer_v, sem_i)
    ci1.wait()
    ci2.wait()
    gathers = []
    for lo, sem in ((0, sem_a), (_HALF, sem_b)):
        hs = pl.ds(lo, _HALF)
        gathers.append((
            pltpu.async_copy(theta_hbm.at[stu_v.at[hs]], th_v.at[hs], sem),
            pltpu.async_copy(a_hbm.at[exer_v.at[hs]], a_v.at[hs], sem),
            pltpu.async_copy(b_hbm.at[exer_v.at[hs]], b_v.at[hs], sem),
        ))
    outs = []
    for half, (g1, g2, g3) in enumerate(gathers):
        g1.wait()
        g2.wait()
        g3.wait()
        lo = half * _HALF
        for j in range(_HALF // _L):
            sl = pl.ds(lo + j * _L, _L)
            th = th_v[sl]
            ar = a_v[sl]
            br = b_v[sl]
            a2 = 2.0 / (1.0 + jnp.exp(-ar))
            z = 1.7 * a2 * (th - br)
            out_v[sl] = 1.0 / (1.0 + jnp.exp(-z))
        hs = pl.ds(lo, _HALF)
        outs.append(pltpu.async_copy(
            out_v.at[hs], out_hbm.at[pl.ds(base + lo, _HALF)], sem_o))
    for o in outs:
        o.wait()


_irt_sc = functools.partial(
    pl.kernel,
    mesh=plsc.VectorSubcoreMesh(core_axis_name="c", subcore_axis_name="s"),
    out_type=jax.ShapeDtypeStruct((BATCH,), jnp.float32),
    scratch_types=[
        pltpu.VMEM((_BPW,), jnp.int32),
        pltpu.VMEM((_BPW,), jnp.int32),
        pltpu.VMEM((_BPW,), jnp.float32),
        pltpu.VMEM((_BPW,), jnp.float32),
        pltpu.VMEM((_BPW,), jnp.float32),
        pltpu.VMEM((_BPW,), jnp.float32),
        pltpu.SemaphoreType.DMA,
        pltpu.SemaphoreType.DMA,
        pltpu.SemaphoreType.DMA,
        pltpu.SemaphoreType.DMA,
    ],
)(_irt_body)


def kernel(stu_id, exer_id, theta_w, a_w, b_w):
    return _irt_sc(
        stu_id.astype(jnp.int32),
        exer_id.astype(jnp.int32),
        theta_w.reshape(-1),
        a_w.reshape(-1),
        b_w.reshape(-1),
    )


# speedup vs baseline: 3.3471x; 2.9127x over previous
import functools

import jax
import jax.numpy as jnp
from jax import lax
from jax.experimental import pallas as pl
from jax.experimental.pallas import tpu as pltpu
from jax.experimental.pallas import tpu_sc as plsc

BATCH = 16384
_INFO = plsc.get_sparse_core_info()
_NC, _NS, _L = _INFO.num_cores, _INFO.num_subcores, _INFO.num_lanes
_NW = _NC * _NS
_BPW = BATCH // _NW
_HALF = _BPW // 2


def _irt_body(stu_hbm, exer_hbm, theta_hbm, a_hbm, b_hbm, out_hbm,
              stu_v, exer_v, th_v, a_v, b_v, out_v,
              sem_i, sem_a, sem_b, sem_o):
    wid = lax.axis_index("s") * _NC + lax.axis_index("c")
    base = wid * _BPW
    ci1 = pltpu.async_copy(stu_hbm.at[pl.ds(base, _BPW)], stu_v.at[0], sem_i)
    ci2 = pltpu.async_copy(exer_hbm.at[pl.ds(base, _BPW)], exer_v.at[0], sem_i)
    ci1.wait()
    ci2.wait()
    gathers = []
    for lo, sem in ((0, sem_a), (_HALF, sem_b)):
        hs = pl.ds(lo, _HALF)
        gathers.append((
            pltpu.async_copy(theta_hbm.at[stu_v.at[:, hs]], th_v.at[:, hs], sem),
            pltpu.async_copy(a_hbm.at[exer_v.at[:, hs]], a_v.at[:, hs], sem),
            pltpu.async_copy(b_hbm.at[exer_v.at[:, hs]], b_v.at[:, hs], sem),
        ))
    outs = []
    for half, (g1, g2, g3) in enumerate(gathers):
        g1.wait()
        g2.wait()
        g3.wait()
        lo = half * _HALF
        for j in range(_HALF // _L):
            sl = pl.ds(lo + j * _L, _L)
            th = th_v[0, sl]
            ar = a_v[0, sl]
            br = b_v[0, sl]
            a2 = 2.0 / (1.0 + jnp.exp(-ar))
            z = 1.7 * a2 * (th - br)
            out_v[sl] = 1.0 / (1.0 + jnp.exp(-z))
        hs = pl.ds(lo, _HALF)
        outs.append(pltpu.async_copy(
            out_v.at[hs], out_hbm.at[pl.ds(base + lo, _HALF)], sem_o))
    for o in outs:
        o.wait()


_irt_sc = functools.partial(
    pl.kernel,
    mesh=plsc.VectorSubcoreMesh(core_axis_name="c", subcore_axis_name="s"),
    out_type=jax.ShapeDtypeStruct((BATCH,), jnp.float32),
    scratch_types=[
        pltpu.VMEM((1, _BPW), jnp.int32),
        pltpu.VMEM((1, _BPW), jnp.int32),
        pltpu.VMEM((1, _BPW), jnp.float32),
        pltpu.VMEM((1, _BPW), jnp.float32),
        pltpu.VMEM((1, _BPW), jnp.float32),
        pltpu.VMEM((_BPW,), jnp.float32),
        pltpu.SemaphoreType.DMA,
        pltpu.SemaphoreType.DMA,
        pltpu.SemaphoreType.DMA,
        pltpu.SemaphoreType.DMA,
    ],
)(_irt_body)


def kernel(stu_id, exer_id, theta_w, a_w, b_w):
    return _irt_sc(
        stu_id.astype(jnp.int32),
        exer_id.astype(jnp.int32),
        theta_w.reshape(1, -1),
        a_w.reshape(1, -1),
        b_w.reshape(1, -1),
    )


# speedup vs baseline: 3.3547x; 1.0023x over previous
import functools

import jax
import jax.numpy as jnp
from jax import lax
from jax.experimental import pallas as pl
from jax.experimental.pallas import tpu as pltpu
from jax.experimental.pallas import tpu_sc as plsc

BATCH = 16384
_INFO = plsc.get_sparse_core_info()
_NC, _NS, _L = _INFO.num_cores, _INFO.num_subcores, _INFO.num_lanes
_NW = _NC * _NS
_BPW = BATCH // _NW
_QTR = _BPW // 4


def _irt_body(stu_hbm, exer_hbm, theta_hbm, a_hbm, b_hbm, out_hbm,
              stu_v, exer_v, th_v, a_v, b_v, out_v,
              sem_i, q0, q1, q2, q3, sem_o):
    sem_q = (q0, q1, q2, q3)
    wid = lax.axis_index("s") * _NC + lax.axis_index("c")
    base = wid * _BPW
    ci1 = pltpu.async_copy(stu_hbm.at[pl.ds(base, _BPW)], stu_v.at[0], sem_i)
    ci2 = pltpu.async_copy(exer_hbm.at[pl.ds(base, _BPW)], exer_v.at[0], sem_i)
    ci1.wait()
    ci2.wait()
    gathers = []
    for q, sem in enumerate(sem_q):
        qsl = pl.ds(q * _QTR, _QTR)
        gathers.append((
            pltpu.async_copy(theta_hbm.at[stu_v.at[:, qsl]], th_v.at[:, qsl], sem),
            pltpu.async_copy(a_hbm.at[exer_v.at[:, qsl]], a_v.at[:, qsl], sem),
            pltpu.async_copy(b_hbm.at[exer_v.at[:, qsl]], b_v.at[:, qsl], sem),
        ))
    outs = []
    for q, (g1, g2, g3) in enumerate(gathers):
        g1.wait()
        g2.wait()
        g3.wait()
        lo = q * _QTR
        for j in range(_QTR // _L):
            sl = pl.ds(lo + j * _L, _L)
            th = th_v[0, sl]
            ar = a_v[0, sl]
            br = b_v[0, sl]
            a2 = 2.0 / (1.0 + jnp.exp(-ar))
            z = 1.7 * a2 * (th - br)
            out_v[sl] = 1.0 / (1.0 + jnp.exp(-z))
        outs.append(pltpu.async_copy(
            out_v.at[pl.ds(lo, _QTR)], out_hbm.at[pl.ds(base + lo, _QTR)], sem_o))
    for o in outs:
        o.wait()


_irt_sc = functools.partial(
    pl.kernel,
    mesh=plsc.VectorSubcoreMesh(core_axis_name="c", subcore_axis_name="s"),
    out_type=jax.ShapeDtypeStruct((BATCH,), jnp.float32),
    scratch_types=[
        pltpu.VMEM((1, _BPW), jnp.int32),
        pltpu.VMEM((1, _BPW), jnp.int32),
        pltpu.VMEM((1, _BPW), jnp.float32),
        pltpu.VMEM((1, _BPW), jnp.float32),
        pltpu.VMEM((1, _BPW), jnp.float32),
        pltpu.VMEM((_BPW,), jnp.float32),
        pltpu.SemaphoreType.DMA,
        pltpu.SemaphoreType.DMA,
        pltpu.SemaphoreType.DMA,
        pltpu.SemaphoreType.DMA,
        pltpu.SemaphoreType.DMA,
        pltpu.SemaphoreType.DMA,
    ],
)(_irt_body)


def kernel(stu_id, exer_id, theta_w, a_w, b_w):
    return _irt_sc(
        stu_id.astype(jnp.int32),
        exer_id.astype(jnp.int32),
        theta_w.reshape(1, -1),
        a_w.reshape(1, -1),
        b_w.reshape(1, -1),
    )


# speedup vs baseline: 3.4496x; 1.0283x over previous
import functools

import jax
import jax.numpy as jnp
from jax import lax
from jax.experimental import pallas as pl
from jax.experimental.pallas import tpu as pltpu
from jax.experimental.pallas import tpu_sc as plsc

BATCH = 16384
_INFO = plsc.get_sparse_core_info()
_NC, _NS, _L = _INFO.num_cores, _INFO.num_subcores, _INFO.num_lanes
_NW = _NC * _NS
_BPW = BATCH // _NW
_QTR = _BPW // 4


def _irt_body(stu_hbm, exer_hbm, theta_hbm, a_hbm, b_hbm, out_hbm,
              stu_v, exer_v, th_v, a_v, b_v, out_v,
              sem_i, q0, q1, q2, q3, sem_o):
    sem_q = (q0, q1, q2, q3)
    wid = lax.axis_index("s") * _NC + lax.axis_index("c")
    base = wid * _BPW
    ci1 = pltpu.async_copy(stu_hbm.at[pl.ds(base, _BPW)], stu_v.at[0], sem_i)
    ci2 = pltpu.async_copy(exer_hbm.at[pl.ds(base, _BPW)], exer_v.at[0], sem_i)
    ci1.wait()
    ci2.wait()
    gathers = []
    for q, sem in enumerate(sem_q):
        qsl = pl.ds(q * _QTR, _QTR)
        gathers.append((
            pltpu.async_copy(theta_hbm.at[stu_v.at[:, qsl]], th_v.at[:, qsl], sem),
            pltpu.async_copy(a_hbm.at[exer_v.at[:, qsl]], a_v.at[:, qsl], sem),
            pltpu.async_copy(b_hbm.at[exer_v.at[:, qsl]], b_v.at[:, qsl], sem),
        ))
    outs = []
    for q, (g1, g2, g3) in enumerate(gathers):
        g1.wait()
        g2.wait()
        g3.wait()
        lo = q * _QTR
        for j in range(_QTR // _L):
            sl = pl.ds(lo + j * _L, _L)
            a_v[0, sl] = jnp.exp(-a_v[0, sl])
        for j in range(_QTR // _L):
            sl = pl.ds(lo + j * _L, _L)
            z = 3.4 * (th_v[0, sl] - b_v[0, sl]) / (1.0 + a_v[0, sl])
            th_v[0, sl] = jnp.exp(-z)
        for j in range(_QTR // _L):
            sl = pl.ds(lo + j * _L, _L)
            out_v[sl] = 1.0 / (1.0 + th_v[0, sl])
        outs.append(pltpu.async_copy(
            out_v.at[pl.ds(lo, _QTR)], out_hbm.at[pl.ds(base + lo, _QTR)], sem_o))
    for o in outs:
        o.wait()


_irt_sc = functools.partial(
    pl.kernel,
    mesh=plsc.VectorSubcoreMesh(core_axis_name="c", subcore_axis_name="s"),
    out_type=jax.ShapeDtypeStruct((BATCH,), jnp.float32),
    scratch_types=[
        pltpu.VMEM((1, _BPW), jnp.int32),
        pltpu.VMEM((1, _BPW), jnp.int32),
        pltpu.VMEM((1, _BPW), jnp.float32),
        pltpu.VMEM((1, _BPW), jnp.float32),
        pltpu.VMEM((1, _BPW), jnp.float32),
        pltpu.VMEM((_BPW,), jnp.float32),
        pltpu.SemaphoreType.DMA,
        pltpu.SemaphoreType.DMA,
        pltpu.SemaphoreType.DMA,
        pltpu.SemaphoreType.DMA,
        pltpu.SemaphoreType.DMA,
        pltpu.SemaphoreType.DMA,
    ],
)(_irt_body)


def kernel(stu_id, exer_id, theta_w, a_w, b_w):
    return _irt_sc(
        stu_id.astype(jnp.int32),
        exer_id.astype(jnp.int32),
        theta_w.reshape(1, -1),
        a_w.reshape(1, -1),
        b_w.reshape(1, -1),
    )


# speedup vs baseline: 3.4556x; 1.0017x over previous
import functools

import jax
import jax.numpy as jnp
from jax import lax
from jax.experimental import pallas as pl
from jax.experimental.pallas import tpu as pltpu
from jax.experimental.pallas import tpu_sc as plsc

BATCH = 16384
_INFO = plsc.get_sparse_core_info()
_NC, _NS, _L = _INFO.num_cores, _INFO.num_subcores, _INFO.num_lanes
_NW = _NC * _NS
_BPW = BATCH // _NW
_QTR = _BPW // 4


def _irt_body(stu_hbm, exer_hbm, theta_hbm, a_hbm, b_hbm, out_hbm,
              stu_v, exer_v, th_v, a_v, b_v, out_v,
              q0, q1, q2, q3, sem_o):
    sem_q = (q0, q1, q2, q3)
    wid = lax.axis_index("s") * _NC + lax.axis_index("c")
    base = wid * _BPW
    idx_copies = []
    for q, sem in enumerate(sem_q):
        qsl = pl.ds(q * _QTR, _QTR)
        hsl = pl.ds(base + q * _QTR, _QTR)
        idx_copies.append((
            pltpu.async_copy(stu_hbm.at[hsl], stu_v.at[0, qsl], sem),
            pltpu.async_copy(exer_hbm.at[hsl], exer_v.at[0, qsl], sem),
        ))
    gathers = []
    for q, sem in enumerate(sem_q):
        i1, i2 = idx_copies[q]
        i1.wait()
        i2.wait()
        qsl = pl.ds(q * _QTR, _QTR)
        gathers.append((
            pltpu.async_copy(theta_hbm.at[stu_v.at[:, qsl]], th_v.at[:, qsl], sem),
            pltpu.async_copy(a_hbm.at[exer_v.at[:, qsl]], a_v.at[:, qsl], sem),
            pltpu.async_copy(b_hbm.at[exer_v.at[:, qsl]], b_v.at[:, qsl], sem),
        ))
    outs = []
    for q, (g1, g2, g3) in enumerate(gathers):
        g1.wait()
        g2.wait()
        g3.wait()
        lo = q * _QTR
        for j in range(_QTR // _L):
            sl = pl.ds(lo + j * _L, _L)
            a_v[0, sl] = jnp.exp(-a_v[0, sl])
        for j in range(_QTR // _L):
            sl = pl.ds(lo + j * _L, _L)
            z = 3.4 * (th_v[0, sl] - b_v[0, sl]) / (1.0 + a_v[0, sl])
            th_v[0, sl] = jnp.exp(-z)
        for j in range(_QTR // _L):
            sl = pl.ds(lo + j * _L, _L)
            out_v[sl] = 1.0 / (1.0 + th_v[0, sl])
        outs.append(pltpu.async_copy(
            out_v.at[pl.ds(lo, _QTR)], out_hbm.at[pl.ds(base + lo, _QTR)], sem_o))
    for o in outs:
        o.wait()


_irt_sc = functools.partial(
    pl.kernel,
    mesh=plsc.VectorSubcoreMesh(core_axis_name="c", subcore_axis_name="s"),
    out_type=jax.ShapeDtypeStruct((BATCH,), jnp.float32),
    scratch_types=[
        pltpu.VMEM((1, _BPW), jnp.int32),
        pltpu.VMEM((1, _BPW), jnp.int32),
        pltpu.VMEM((1, _BPW), jnp.float32),
        pltpu.VMEM((1, _BPW), jnp.float32),
        pltpu.VMEM((1, _BPW), jnp.float32),
        pltpu.VMEM((_BPW,), jnp.float32),
        pltpu.SemaphoreType.DMA,
        pltpu.SemaphoreType.DMA,
        pltpu.SemaphoreType.DMA,
        pltpu.SemaphoreType.DMA,
        pltpu.SemaphoreType.DMA,
    ],
)(_irt_body)


def kernel(stu_id, exer_id, theta_w, a_w, b_w):
    return _irt_sc(
        stu_id.astype(jnp.int32),
        exer_id.astype(jnp.int32),
        theta_w.reshape(1, -1),
        a_w.reshape(1, -1),
        b_w.reshape(1, -1),
    )


# speedup vs baseline: 3.4672x; 1.0034x over previous
import functools

import jax
import jax.numpy as jnp
from jax import lax
from jax.experimental import pallas as pl
from jax.experimental.pallas import tpu as pltpu
from jax.experimental.pallas import tpu_sc as plsc

BATCH = 16384
_INFO = plsc.get_sparse_core_info()
_NC, _NS, _L = _INFO.num_cores, _INFO.num_subcores, _INFO.num_lanes
_NW = _NC * _NS
_BPW = BATCH // _NW
_QTR = _BPW // 4


def _irt_body(stu_hbm, exer_hbm, theta_hbm, a_hbm, b_hbm, out_hbm,
              stu_v, exer_v, th_v, a_v, b_v, out_v,
              q0, q1, q2, q3, sem_o):
    sem_q = (q0, q1, q2, q3)
    wid = lax.axis_index("s") * _NC + lax.axis_index("c")
    base = wid * _BPW
    idx_copies = []
    for q, sem in enumerate(sem_q):
        qsl = pl.ds(q * _QTR, _QTR)
        hsl = pl.ds(base + q * _QTR, _QTR)
        idx_copies.append((
            pltpu.async_copy(stu_hbm.at[hsl], stu_v.at[0, qsl], sem),
            pltpu.async_copy(exer_hbm.at[hsl], exer_v.at[0, qsl], sem),
        ))
    gathers = []
    for q, sem in enumerate(sem_q):
        i1, i2 = idx_copies[q]
        i1.wait()
        i2.wait()
        qsl = pl.ds(q * _QTR, _QTR)
        gathers.append((
            pltpu.async_copy(a_hbm.at[exer_v.at[:, qsl]], a_v.at[:, qsl], sem),
            pltpu.async_copy(theta_hbm.at[stu_v.at[:, qsl]], th_v.at[:, qsl], sem),
            pltpu.async_copy(b_hbm.at[exer_v.at[:, qsl]], b_v.at[:, qsl], sem),
        ))
    outs = []
    for q, (ga, gth, gb) in enumerate(gathers):
        lo = q * _QTR
        ga.wait()
        for j in range(_QTR // _L):
            sl = pl.ds(lo + j * _L, _L)
            a_v[0, sl] = jnp.exp(-a_v[0, sl])
        gth.wait()
        gb.wait()
        for j in range(_QTR // _L):
            sl = pl.ds(lo + j * _L, _L)
            z = 3.4 * (th_v[0, sl] - b_v[0, sl]) / (1.0 + a_v[0, sl])
            th_v[0, sl] = jnp.exp(-z)
        for j in range(_QTR // _L):
            sl = pl.ds(lo + j * _L, _L)
            out_v[sl] = 1.0 / (1.0 + th_v[0, sl])
        outs.append(pltpu.async_copy(
            out_v.at[pl.ds(lo, _QTR)], out_hbm.at[pl.ds(base + lo, _QTR)], sem_o))
    for o in outs:
        o.wait()


_irt_sc = functools.partial(
    pl.kernel,
    mesh=plsc.VectorSubcoreMesh(core_axis_name="c", subcore_axis_name="s"),
    out_type=jax.ShapeDtypeStruct((BATCH,), jnp.float32),
    scratch_types=[
        pltpu.VMEM((1, _BPW), jnp.int32),
        pltpu.VMEM((1, _BPW), jnp.int32),
        pltpu.VMEM((1, _BPW), jnp.float32),
        pltpu.VMEM((1, _BPW), jnp.float32),
        pltpu.VMEM((1, _BPW), jnp.float32),
        pltpu.VMEM((_BPW,), jnp.float32),
        pltpu.SemaphoreType.DMA,
        pltpu.SemaphoreType.DMA,
        pltpu.SemaphoreType.DMA,
        pltpu.SemaphoreType.DMA,
        pltpu.SemaphoreType.DMA,
    ],
)(_irt_body)


def kernel(stu_id, exer_id, theta_w, a_w, b_w):
    return _irt_sc(
        stu_id.astype(jnp.int32),
        exer_id.astype(jnp.int32),
        theta_w.reshape(1, -1),
        a_w.reshape(1, -1),
        b_w.reshape(1, -1),
    )
